# baseline (speedup 1.0000x reference)
import jax
import jax.numpy as jnp
from jax.experimental import pallas as pl

_D_MODEL = 2048
_BLK = 512


def _onehot_body(ids_ref, out_ref):
    ids = ids_ref[0]
    iota = jax.lax.broadcasted_iota(jnp.int32, (_BLK, _D_MODEL), 1)
    out_ref[...] = jnp.where(iota == ids, 1.0, 0.0).astype(jnp.float32)


def kernel(input_ids):
    b, s = input_ids.shape
    ids = input_ids.astype(jnp.int32)
    padded = jnp.concatenate([jnp.zeros((b, 1), jnp.int32), ids], axis=1)
    n = b * (s + 1)
    nb = (n + _BLK - 1) // _BLK
    flat = jnp.pad(padded.reshape(-1), (0, nb * _BLK - n),
                   constant_values=_D_MODEL)
    ids3 = flat.reshape(nb, _BLK, 1)
    out = pl.pallas_call(
        _onehot_body,
        grid=(nb,),
        in_specs=[pl.BlockSpec((1, _BLK, 1), lambda i: (i, 0, 0))],
        out_specs=pl.BlockSpec((_BLK, _D_MODEL), lambda i: (i, 0)),
        out_shape=jax.ShapeDtypeStruct((n, _D_MODEL), jnp.float32),
    )(ids3)
    return out.reshape(b, s + 1, _D_MODEL)


# speedup vs baseline: 1.0714x; 1.0714x over previous
import jax
import jax.numpy as jnp
from jax.experimental import pallas as pl

_D_MODEL = 2048
_BLK = 512


def _onehot_body(ids_ref, out_ref):
    ids = ids_ref[0, 0]
    iota = jax.lax.broadcasted_iota(jnp.int32, (_BLK, _D_MODEL), 1)
    out_ref[0] = jnp.where(iota == ids, 1.0, 0.0).astype(jnp.float32)


def kernel(input_ids):
    b, s = input_ids.shape
    ids = input_ids.astype(jnp.int32)
    padded = jnp.concatenate([jnp.zeros((b, 1), jnp.int32), ids], axis=1)
    sp = s + 1
    nb = (sp + _BLK - 1) // _BLK
    flat = jnp.pad(padded, ((0, 0), (0, nb * _BLK - sp)),
                   constant_values=_D_MODEL)
    ids4 = flat.reshape(b, nb, _BLK, 1)
    return pl.pallas_call(
        _onehot_body,
        grid=(b, nb),
        in_specs=[pl.BlockSpec((1, 1, _BLK, 1), lambda i, j: (i, j, 0, 0))],
        out_specs=pl.BlockSpec((1, _BLK, _D_MODEL), lambda i, j: (i, j, 0)),
        out_shape=jax.ShapeDtypeStruct((b, sp, _D_MODEL), jnp.float32),
    )(ids4)
